# baseline (speedup 1.0000x reference)
import jax
import jax.numpy as jnp
from jax.experimental import pallas as pl

_NQ = 4
_K = 1024
_D = 256
_TILE = 1024
_LOSS_W = 1.25


def _rsum_like_baseline(r):
    sq = r * r
    v = sq[:, :128] + sq[:, 128:]
    acc = v[:, 0:8]
    for j in range(1, 16):
        acc = acc + v[:, 8 * j:8 * j + 8]
    h = acc[:, 0:4] + acc[:, 4:8]
    h2 = h[:, 0:2] + h[:, 2:4]
    return h2[:, 0:1] + h2[:, 1:2]


def _fused_kernel(flat_ref, cb_ref, limbs_ref, cb2_ref,
                  q_ref, i0_ref, i1_ref, i2_ref, i3_ref, sse_ref):
    idx_refs = (i0_ref, i1_ref, i2_ref, i3_ref)
    i = pl.program_id(0)

    @pl.when(i == 0)
    def _init():
        sse_ref[...] = jnp.zeros((1, 1), jnp.float32)

    orig = flat_ref[...]
    r = orig
    lane = jax.lax.broadcasted_iota(jnp.int32, (_TILE, _K), 1)
    sse = jnp.zeros((1, 1), jnp.float32)
    for s in range(_NQ):
        rsum = _rsum_like_baseline(r)
        m = jax.lax.dot_general(r, cb_ref[s], (((1,), (1,)), ((), ())),
                                preferred_element_type=jnp.float32)
        d2 = (rsum - 2.0 * m) + cb2_ref[...][:, s * _K:(s + 1) * _K]
        dmin = jnp.min(d2, axis=1, keepdims=True)
        idx = jnp.min(jnp.where(d2 == dmin, lane, _K), axis=1)
        idx = idx.astype(jnp.int32)
        onehot = (lane == idx[:, None]).astype(jnp.float32)
        q3 = jax.lax.dot_general(onehot, limbs_ref[s],
                                 (((1,), (0,)), ((), ())),
                                 preferred_element_type=jnp.float32)
        q = (q3[:, :_D] + q3[:, _D:2 * _D]) + q3[:, 2 * _D:]
        diff = q - r
        sse = sse + jnp.reshape(jnp.sum(diff * diff), (1, 1))
        r = r - q
        idx_refs[s][...] = idx[:, None]
    q_ref[...] = orig - r
    sse_ref[...] = sse_ref[...] + sse


def _trunc_bf16(v):
    return jax.lax.bitcast_convert_type(
        jax.lax.bitcast_convert_type(v, jnp.uint32) & jnp.uint32(0xFFFF0000),
        jnp.float32)


def _split_limbs(cb):
    hi = _trunc_bf16(cb)
    rem = cb - hi
    mid = _trunc_bf16(rem)
    lo = rem - mid
    return jnp.concatenate([hi, mid, lo], axis=1)


def kernel(x, codebooks):
    B, T, D = x.shape
    nq = codebooks.shape[0]
    N = B * T
    flat = x.reshape(N, D)

    limbs = jax.vmap(_split_limbs)(codebooks)
    cb2 = jnp.sum(codebooks ** 2, axis=2)
    cb2_row = cb2.reshape(1, nq * _K)

    quantized_flat, idx0, idx1, idx2, idx3, sse = pl.pallas_call(
        _fused_kernel,
        grid=(N // _TILE,),
        in_specs=[
            pl.BlockSpec((_TILE, D), lambda i: (i, 0)),
            pl.BlockSpec((nq, _K, D), lambda i: (0, 0, 0)),
            pl.BlockSpec((nq, _K, 3 * D), lambda i: (0, 0, 0)),
            pl.BlockSpec((1, nq * _K), lambda i: (0, 0)),
        ],
        out_specs=[
            pl.BlockSpec((_TILE, D), lambda i: (i, 0)),
            pl.BlockSpec((_TILE, 1), lambda i: (i, 0)),
            pl.BlockSpec((_TILE, 1), lambda i: (i, 0)),
            pl.BlockSpec((_TILE, 1), lambda i: (i, 0)),
            pl.BlockSpec((_TILE, 1), lambda i: (i, 0)),
            pl.BlockSpec((1, 1), lambda i: (0, 0)),
        ],
        out_shape=[
            jax.ShapeDtypeStruct((N, D), jnp.float32),
            jax.ShapeDtypeStruct((N, 1), jnp.int32),
            jax.ShapeDtypeStruct((N, 1), jnp.int32),
            jax.ShapeDtypeStruct((N, 1), jnp.int32),
            jax.ShapeDtypeStruct((N, 1), jnp.int32),
            jax.ShapeDtypeStruct((1, 1), jnp.float32),
        ],
    )(flat, codebooks, limbs, cb2_row)

    quantized = quantized_flat.reshape(B, T, D)
    indices = jnp.concatenate([idx0, idx1, idx2, idx3],
                              axis=1).reshape(B, T, nq)
    vq_loss = sse[0, 0] * (_LOSS_W / (N * D))
    losses = jnp.full((nq,), vq_loss, dtype=jnp.float32)
    return (quantized, indices, losses)
